# baseline (speedup 1.0000x reference)
import functools

import jax
import jax.numpy as jnp
from jax import lax
from jax.experimental import pallas as pl
from jax.experimental.pallas import tpu as pltpu
from jax.experimental.pallas import tpu_sc as plsc

SCALE = 8.0


def kernel(X, table):
    R, S = X.shape
    V, D = table.shape
    RB = R // 128
    n_tasks = S * RB

    info = plsc.get_sparse_core_info()
    NC, NS = info.num_cores, info.num_subcores
    NW = NC * NS
    per_w = n_tasks // NW
    CH = 128
    NBUF = 4
    A = 3
    assert per_w % NBUF == 0

    idx = X.T.reshape(n_tasks, CH).astype(jnp.int32)

    tablep = jnp.broadcast_to(table[:, None, :], (V, 2, D)).reshape(V, 2 * D)
    table16 = tablep.reshape(16 * V, 8)

    mesh = plsc.VectorSubcoreMesh(core_axis_name="c", subcore_axis_name="s")

    @functools.partial(
        pl.kernel,
        mesh=mesh,
        out_type=jax.ShapeDtypeStruct((S, D // 8, RB, 8, 128), jnp.float32),
        compiler_params=pltpu.CompilerParams(
            use_tc_tiling_on_sc=False, needs_layout_passes=False
        ),
        scratch_types=[
            pltpu.VMEM((per_w, CH), jnp.int32),
            pltpu.VMEM((NBUF, 8, CH), jnp.int32),
            pltpu.VMEM((NBUF, 8, CH, 8), jnp.float32),
            pltpu.VMEM((NBUF, 8, 8, 128), jnp.float32),
            pltpu.SemaphoreType.DMA((NBUF,)),
            pltpu.SemaphoreType.DMA((NBUF,)),
        ],
    )
    def sc_kernel(idx_hbm, tab_hbm, out_hbm, idx_v, ebuf, dst8, tbuf, gsem, ssem):
        wid = lax.axis_index("s") * NC + lax.axis_index("c")
        t0 = wid * per_w
        pltpu.sync_copy(idx_hbm.at[pl.ds(t0, per_w)], idx_v)

        iota = lax.iota(jnp.int32, 16)
        tvecs = [iota + (16 * k) for k in range(8)]

        def build_and_fire(task, slot):
            sh = [idx_v[task, pl.ds(16 * g, 16)] * 16 for g in range(8)]
            for c in range(8):
                for g in range(8):
                    ebuf[slot, c, pl.ds(16 * g, 16)] = sh[g] + c
            for c in range(8):
                pltpu.async_copy(
                    tab_hbm.at[ebuf.at[slot, c]], dst8.at[slot, c], gsem.at[slot]
                )

        for c in range(A):
            build_and_fire(c, c)

        @pl.loop(0, per_w, step=NBUF)
        def outer(j):
            for b in range(NBUF):
                jj = j + b
                nxt = jj + A
                b2 = (b + A) % NBUF

                @pl.when(nxt < per_w)
                def _fire_gather():
                    build_and_fire(nxt, b2)

                for c in range(8):
                    pltpu.make_async_copy(
                        tab_hbm.at[ebuf.at[0, 0]], dst8.at[b, c], gsem.at[b]
                    ).wait()

                @pl.when(jj >= NBUF)
                def _drain_store():
                    pltpu.make_async_copy(
                        tbuf.at[b],
                        out_hbm.at[0, pl.ds(0, 8), 0],
                        ssem.at[b],
                    ).wait()

                @plsc.parallel_loop(0, D, unroll=4)
                def _tr(j5):
                    c = j5 >> 3
                    jl = j5 & 7
                    cv = jnp.full((16,), 0, jnp.int32) + c
                    jv = jnp.full((16,), 0, jnp.int32) + jl
                    for k in range(8):
                        v = plsc.load_gather(dst8.at[b], [cv, tvecs[k], jv])
                        tbuf[b, c, jl, pl.ds(16 * k, 16)] = v * SCALE

                t = t0 + jj
                s = t // RB
                rb = lax.rem(t, RB)
                pltpu.async_copy(
                    tbuf.at[b], out_hbm.at[s, pl.ds(0, 8), rb], ssem.at[b]
                )

        for b in range(NBUF):
            pltpu.make_async_copy(
                tbuf.at[b], out_hbm.at[0, pl.ds(0, 8), 0], ssem.at[b]
            ).wait()

    out5 = sc_kernel(idx, table16)
    return out5.transpose(2, 4, 0, 1, 3).reshape(R, S, D)


# speedup vs baseline: 3.9122x; 3.9122x over previous
import functools

import jax
import jax.numpy as jnp
from jax import lax
from jax.experimental import pallas as pl
from jax.experimental.pallas import tpu as pltpu
from jax.experimental.pallas import tpu_sc as plsc

SCALE = 8.0


def kernel(X, table):
    R, S = X.shape
    V, D = table.shape
    RB = R // 128
    n_tasks = S * RB

    info = plsc.get_sparse_core_info()
    NC, NS = info.num_cores, info.num_subcores
    NW = NC * NS
    per_w = n_tasks // NW
    CH = 128
    NBUF = 4
    A = 3
    assert per_w % NBUF == 0

    idx = X.T.reshape(n_tasks, CH).astype(jnp.int32)

    tablep = jnp.pad(table, ((0, 0), (0, 128 - D)))
    table16 = tablep.reshape(16 * V, 8)

    mesh = plsc.VectorSubcoreMesh(core_axis_name="c", subcore_axis_name="s")

    @functools.partial(
        pl.kernel,
        mesh=mesh,
        out_type=jax.ShapeDtypeStruct((S, D // 8, RB, 8, 128), jnp.float32),
        compiler_params=pltpu.CompilerParams(
            use_tc_tiling_on_sc=False, needs_layout_passes=False
        ),
        scratch_types=[
            pltpu.VMEM((per_w, CH), jnp.int32),
            pltpu.VMEM((NBUF, 8, CH), jnp.int32),
            pltpu.VMEM((NBUF, 8, CH, 8), jnp.float32),
            pltpu.VMEM((NBUF, 8, 8, 128), jnp.float32),
            pltpu.SemaphoreType.DMA((NBUF,)),
            pltpu.SemaphoreType.DMA((NBUF,)),
        ],
    )
    def sc_kernel(idx_hbm, tab_hbm, out_hbm, idx_v, ebuf, dst8, tbuf, gsem, ssem):
        wid = lax.axis_index("s") * NC + lax.axis_index("c")
        t0 = wid * per_w
        pltpu.sync_copy(idx_hbm.at[pl.ds(t0, per_w)], idx_v)

        iota = lax.iota(jnp.int32, 16)
        tvecs = [iota + (16 * k) for k in range(8)]

        def build_and_fire(task, slot):
            sh = [idx_v[task, pl.ds(16 * g, 16)] * 16 for g in range(8)]
            for c in range(8):
                for g in range(8):
                    ebuf[slot, c, pl.ds(16 * g, 16)] = sh[g] + c
            for c in range(8):
                pltpu.async_copy(
                    tab_hbm.at[ebuf.at[slot, c]], dst8.at[slot, c], gsem.at[slot]
                )

        for c in range(A):
            build_and_fire(c, c)

        @pl.loop(0, per_w, step=NBUF)
        def outer(j):
            for b in range(NBUF):
                jj = j + b
                nxt = jj + A
                b2 = (b + A) % NBUF

                @pl.when(nxt < per_w)
                def _fire_gather():
                    build_and_fire(nxt, b2)

                for c in range(8):
                    pltpu.make_async_copy(
                        tab_hbm.at[ebuf.at[0, 0]], dst8.at[b, c], gsem.at[b]
                    ).wait()

                @pl.when(jj >= NBUF)
                def _drain_store():
                    pltpu.make_async_copy(
                        tbuf.at[b],
                        out_hbm.at[0, pl.ds(0, 8), 0],
                        ssem.at[b],
                    ).wait()

                @plsc.parallel_loop(0, D, unroll=4)
                def _tr(j5):
                    c = j5 >> 3
                    jl = j5 & 7
                    cv = jnp.full((16,), 0, jnp.int32) + c
                    jv = jnp.full((16,), 0, jnp.int32) + jl
                    for k in range(8):
                        v = plsc.load_gather(dst8.at[b], [cv, tvecs[k], jv])
                        tbuf[b, c, jl, pl.ds(16 * k, 16)] = v * SCALE

                t = t0 + jj
                s = t // RB
                rb = lax.rem(t, RB)
                pltpu.async_copy(
                    tbuf.at[b], out_hbm.at[s, pl.ds(0, 8), rb], ssem.at[b]
                )

        for b in range(NBUF):
            pltpu.make_async_copy(
                tbuf.at[b], out_hbm.at[0, pl.ds(0, 8), 0], ssem.at[b]
            ).wait()

    out5 = sc_kernel(idx, table16)
    return out5.transpose(2, 4, 0, 1, 3).reshape(R, S, D)


# speedup vs baseline: 5.7578x; 1.4718x over previous
import functools

import jax
import jax.numpy as jnp
from jax import lax
from jax.experimental import pallas as pl
from jax.experimental.pallas import tpu as pltpu
from jax.experimental.pallas import tpu_sc as plsc

SCALE = 8.0


def kernel(X, table):
    R, S = X.shape
    V, D = table.shape
    RB = R // 128
    n_tasks = S * RB

    info = plsc.get_sparse_core_info()
    NC, NS = info.num_cores, info.num_subcores
    NW = NC * NS
    per_w = n_tasks // NW
    CH = 128
    NBUF = 4
    A = 3
    assert per_w % NBUF == 0

    idx = X.T.reshape(n_tasks, CH).astype(jnp.int32)

    BK = 8192
    nblk = (V + BK - 1) // BK

    def tc_widen_body(tt_ref, out_ref):
        x = tt_ref[...]
        out_ref[:, :D] = x.T * SCALE
        out_ref[:, D:] = jnp.zeros((BK, 128 - D), jnp.float32)

    tc_widen = pl.pallas_call(
        tc_widen_body,
        grid=(nblk,),
        in_specs=[pl.BlockSpec((D, BK), lambda i: (0, i))],
        out_specs=pl.BlockSpec((BK, 128), lambda i: (i, 0)),
        out_shape=jax.ShapeDtypeStruct((V, 128), jnp.float32),
    )
    table16 = tc_widen(table.T).reshape(16 * V, 8)

    mesh = plsc.VectorSubcoreMesh(core_axis_name="c", subcore_axis_name="s")

    @functools.partial(
        pl.kernel,
        mesh=mesh,
        out_type=jax.ShapeDtypeStruct((S, D // 8, RB, 8, 128), jnp.float32),
        compiler_params=pltpu.CompilerParams(
            use_tc_tiling_on_sc=False, needs_layout_passes=False
        ),
        scratch_types=[
            pltpu.VMEM((per_w, CH), jnp.int32),
            pltpu.VMEM((NBUF, 8, CH), jnp.int32),
            pltpu.VMEM((NBUF, 8, CH, 8), jnp.float32),
            pltpu.VMEM((NBUF, 8, 8, 128), jnp.float32),
            pltpu.SemaphoreType.DMA((NBUF,)),
            pltpu.SemaphoreType.DMA((NBUF,)),
        ],
    )
    def sc_kernel(idx_hbm, tab_hbm, out_hbm, idx_v, ebuf, dst8, tbuf, gsem, ssem):
        wid = lax.axis_index("s") * NC + lax.axis_index("c")
        t0 = wid * per_w
        pltpu.sync_copy(idx_hbm.at[pl.ds(t0, per_w)], idx_v)

        iota = lax.iota(jnp.int32, 16)
        tvecs = [iota + (16 * k) for k in range(8)]

        def build_and_fire(task, slot):
            sh = [idx_v[task, pl.ds(16 * g, 16)] * 16 for g in range(8)]
            for c in range(8):
                for g in range(8):
                    ebuf[slot, c, pl.ds(16 * g, 16)] = sh[g] + c
            for c in range(8):
                pltpu.async_copy(
                    tab_hbm.at[ebuf.at[slot, c]], dst8.at[slot, c], gsem.at[slot]
                )

        for c in range(A):
            build_and_fire(c, c)

        @pl.loop(0, per_w, step=NBUF)
        def outer(j):
            for b in range(NBUF):
                jj = j + b
                nxt = jj + A
                b2 = (b + A) % NBUF

                @pl.when(nxt < per_w)
                def _fire_gather():
                    build_and_fire(nxt, b2)

                for c in range(8):
                    pltpu.make_async_copy(
                        tab_hbm.at[ebuf.at[0, 0]], dst8.at[b, c], gsem.at[b]
                    ).wait()

                @pl.when(jj >= NBUF)
                def _drain_store():
                    pltpu.make_async_copy(
                        tbuf.at[b],
                        out_hbm.at[0, pl.ds(0, 8), 0],
                        ssem.at[b],
                    ).wait()

                @plsc.parallel_loop(0, D, unroll=4)
                def _tr(j5):
                    c = j5 >> 3
                    jl = j5 & 7
                    cv = jnp.full((16,), 0, jnp.int32) + c
                    jv = jnp.full((16,), 0, jnp.int32) + jl
                    for k in range(8):
                        v = plsc.load_gather(dst8.at[b], [cv, tvecs[k], jv])
                        tbuf[b, c, jl, pl.ds(16 * k, 16)] = v

                t = t0 + jj
                s = t // RB
                rb = lax.rem(t, RB)
                pltpu.async_copy(
                    tbuf.at[b], out_hbm.at[s, pl.ds(0, 8), rb], ssem.at[b]
                )

        for b in range(NBUF):
            pltpu.make_async_copy(
                tbuf.at[b], out_hbm.at[0, pl.ds(0, 8), 0], ssem.at[b]
            ).wait()

    out5 = sc_kernel(idx, table16)
    return out5.transpose(2, 4, 0, 1, 3).reshape(R, S, D)


# speedup vs baseline: 6.6987x; 1.1634x over previous
import functools

import jax
import jax.numpy as jnp
from jax import lax
from jax.experimental import pallas as pl
from jax.experimental.pallas import tpu as pltpu
from jax.experimental.pallas import tpu_sc as plsc

SCALE = 8.0


def kernel(X, table):
    R, S = X.shape
    V, D = table.shape
    RB = R // 128
    n_tasks = S * RB

    info = plsc.get_sparse_core_info()
    NC, NS = info.num_cores, info.num_subcores
    NW = NC * NS
    per_w = n_tasks // NW
    CH = 128
    NBUF = 4
    A = 3
    assert per_w % NBUF == 0

    idx = X.T.reshape(n_tasks, CH).astype(jnp.int32)

    BK = 8192
    nblk = (V + BK - 1) // BK

    def tc_widen_body(tt_ref, out_ref):
        x = tt_ref[...]
        out_ref[:, :D] = x.T * SCALE
        out_ref[:, D:] = jnp.zeros((BK, 128 - D), jnp.float32)

    tc_widen = pl.pallas_call(
        tc_widen_body,
        grid=(nblk,),
        in_specs=[pl.BlockSpec((D, BK), lambda i: (0, i))],
        out_specs=pl.BlockSpec((BK, 128), lambda i: (i, 0)),
        out_shape=jax.ShapeDtypeStruct((V, 128), jnp.float32),
    )
    table16 = tc_widen(table.T).reshape(8 * V, 16)

    mesh = plsc.VectorSubcoreMesh(core_axis_name="c", subcore_axis_name="s")

    @functools.partial(
        pl.kernel,
        mesh=mesh,
        out_type=jax.ShapeDtypeStruct((S, D // 8, RB, 8, 128), jnp.float32),
        compiler_params=pltpu.CompilerParams(
            use_tc_tiling_on_sc=False, needs_layout_passes=False
        ),
        scratch_types=[
            pltpu.VMEM((per_w, CH), jnp.int32),
            pltpu.VMEM((NBUF, 4, CH), jnp.int32),
            pltpu.VMEM((NBUF, 4, CH, 16), jnp.float32),
            pltpu.VMEM((NBUF, 8, 8, 128), jnp.float32),
            pltpu.SemaphoreType.DMA((NBUF,)),
            pltpu.SemaphoreType.DMA((NBUF,)),
        ],
    )
    def sc_kernel(idx_hbm, tab_hbm, out_hbm, idx_v, ebuf, dst8, tbuf, gsem, ssem):
        wid = lax.axis_index("s") * NC + lax.axis_index("c")
        t0 = wid * per_w
        pltpu.sync_copy(idx_hbm.at[pl.ds(t0, per_w)], idx_v)

        iota = lax.iota(jnp.int32, 16)
        tvecs = [iota + (16 * k) for k in range(8)]

        def build_and_fire(task, slot):
            sh = [idx_v[task, pl.ds(16 * g, 16)] * 8 for g in range(8)]
            for c in range(4):
                for g in range(8):
                    ebuf[slot, c, pl.ds(16 * g, 16)] = sh[g] + c
            for c in range(4):
                pltpu.async_copy(
                    tab_hbm.at[ebuf.at[slot, c]], dst8.at[slot, c], gsem.at[slot]
                )

        for c in range(A):
            build_and_fire(c, c)

        @pl.loop(0, per_w, step=NBUF)
        def outer(j):
            for b in range(NBUF):
                jj = j + b
                nxt = jj + A
                b2 = (b + A) % NBUF

                @pl.when(nxt < per_w)
                def _fire_gather():
                    build_and_fire(nxt, b2)

                for c in range(4):
                    pltpu.make_async_copy(
                        tab_hbm.at[ebuf.at[0, 0]], dst8.at[b, c], gsem.at[b]
                    ).wait()

                @pl.when(jj >= NBUF)
                def _drain_store():
                    pltpu.make_async_copy(
                        tbuf.at[b],
                        out_hbm.at[0, pl.ds(0, 8), 0],
                        ssem.at[b],
                    ).wait()

                @plsc.parallel_loop(0, D, unroll=4)
                def _tr(j5):
                    c16 = j5 >> 4
                    jl16 = j5 & 15
                    jh = j5 >> 3
                    jl = j5 & 7
                    cv = jnp.full((16,), 0, jnp.int32) + c16
                    jv = jnp.full((16,), 0, jnp.int32) + jl16
                    for k in range(8):
                        v = plsc.load_gather(dst8.at[b], [cv, tvecs[k], jv])
                        tbuf[b, jh, jl, pl.ds(16 * k, 16)] = v

                t = t0 + jj
                s = t // RB
                rb = lax.rem(t, RB)
                pltpu.async_copy(
                    tbuf.at[b], out_hbm.at[s, pl.ds(0, 8), rb], ssem.at[b]
                )

        for b in range(NBUF):
            pltpu.make_async_copy(
                tbuf.at[b], out_hbm.at[0, pl.ds(0, 8), 0], ssem.at[b]
            ).wait()

    out5 = sc_kernel(idx, table16)
    return out5.transpose(2, 4, 0, 1, 3).reshape(R, S, D)


# speedup vs baseline: 7.0064x; 1.0459x over previous
import functools

import jax
import jax.numpy as jnp
from jax import lax
from jax.experimental import pallas as pl
from jax.experimental.pallas import tpu as pltpu
from jax.experimental.pallas import tpu_sc as plsc

SCALE = 8.0


def kernel(X, table):
    R, S = X.shape
    V, D = table.shape
    RB = R // 128
    n_tasks = S * RB

    info = plsc.get_sparse_core_info()
    NC, NS = info.num_cores, info.num_subcores
    NW = NC * NS
    per_w = n_tasks // NW
    CH = 128
    NBUF = 4
    A = 3
    assert per_w % NBUF == 0

    idx = X.T.reshape(n_tasks, CH).astype(jnp.int32)

    BK = 16384
    nblk = (V + BK - 1) // BK

    def tc_widen_body(tt_ref, out_ref):
        x = tt_ref[...]
        out_ref[:, :D] = x.T * SCALE
        out_ref[:, D:] = jnp.zeros((BK, 128 - D), jnp.float32)

    tc_widen = pl.pallas_call(
        tc_widen_body,
        grid=(nblk,),
        in_specs=[pl.BlockSpec((D, BK), lambda i: (0, i))],
        out_specs=pl.BlockSpec((BK, 128), lambda i: (i, 0)),
        out_shape=jax.ShapeDtypeStruct((V, 128), jnp.float32),
    )
    table16 = tc_widen(table.T).reshape(8 * V, 16)

    mesh = plsc.VectorSubcoreMesh(core_axis_name="c", subcore_axis_name="s")

    @functools.partial(
        pl.kernel,
        mesh=mesh,
        out_type=jax.ShapeDtypeStruct((S, D // 8, RB, 8, 128), jnp.float32),
        compiler_params=pltpu.CompilerParams(
            use_tc_tiling_on_sc=False, needs_layout_passes=False
        ),
        scratch_types=[
            pltpu.VMEM((per_w, CH), jnp.int32),
            pltpu.VMEM((NBUF, 4, CH), jnp.int32),
            pltpu.VMEM((NBUF, 4, CH, 16), jnp.float32),
            pltpu.VMEM((NBUF, 8, 8, 128), jnp.float32),
            pltpu.SemaphoreType.DMA((NBUF,)),
            pltpu.SemaphoreType.DMA((NBUF,)),
        ],
    )
    def sc_kernel(idx_hbm, tab_hbm, out_hbm, idx_v, ebuf, dst8, tbuf, gsem, ssem):
        wid = lax.axis_index("s") * NC + lax.axis_index("c")
        t0 = wid * per_w
        pltpu.sync_copy(idx_hbm.at[pl.ds(t0, per_w)], idx_v)

        iota = lax.iota(jnp.int32, 16)
        tvecs = [iota + (16 * k) for k in range(8)]

        def build_and_fire(task, slot):
            sh = [idx_v[task, pl.ds(16 * g, 16)] * 8 for g in range(8)]
            for c in range(4):
                for g in range(8):
                    ebuf[slot, c, pl.ds(16 * g, 16)] = sh[g] + c
            for c in range(4):
                pltpu.async_copy(
                    tab_hbm.at[ebuf.at[slot, c]], dst8.at[slot, c], gsem.at[slot]
                )

        for c in range(A):
            build_and_fire(c, c)

        @pl.loop(0, per_w, step=NBUF)
        def outer(j):
            for b in range(NBUF):
                jj = j + b
                nxt = jj + A
                b2 = (b + A) % NBUF

                @pl.when(nxt < per_w)
                def _fire_gather():
                    build_and_fire(nxt, b2)

                for c in range(4):
                    pltpu.make_async_copy(
                        tab_hbm.at[ebuf.at[0, 0]], dst8.at[b, c], gsem.at[b]
                    ).wait()

                @pl.when(jj >= NBUF)
                def _drain_store():
                    pltpu.make_async_copy(
                        tbuf.at[b],
                        out_hbm.at[0, pl.ds(0, 8), 0],
                        ssem.at[b],
                    ).wait()

                @plsc.parallel_loop(0, D, unroll=4)
                def _tr(j5):
                    c16 = j5 >> 4
                    jl16 = j5 & 15
                    jh = j5 >> 3
                    jl = j5 & 7
                    cv = jnp.full((16,), 0, jnp.int32) + c16
                    jv = jnp.full((16,), 0, jnp.int32) + jl16
                    for k in range(8):
                        v = plsc.load_gather(dst8.at[b], [cv, tvecs[k], jv])
                        tbuf[b, jh, jl, pl.ds(16 * k, 16)] = v

                t = t0 + jj
                s = t // RB
                rb = lax.rem(t, RB)
                pltpu.async_copy(
                    tbuf.at[b], out_hbm.at[s, pl.ds(0, 8), rb], ssem.at[b]
                )

        for b in range(NBUF):
            pltpu.make_async_copy(
                tbuf.at[b], out_hbm.at[0, pl.ds(0, 8), 0], ssem.at[b]
            ).wait()

    out5 = sc_kernel(idx, table16)
    return out5.transpose(2, 4, 0, 1, 3).reshape(R, S, D)


# speedup vs baseline: 7.1152x; 1.0155x over previous
import functools

import jax
import jax.numpy as jnp
from jax import lax
from jax.experimental import pallas as pl
from jax.experimental.pallas import tpu as pltpu
from jax.experimental.pallas import tpu_sc as plsc

SCALE = 8.0


def kernel(X, table):
    R, S = X.shape
    V, D = table.shape
    RB = R // 128
    n_tasks = S * RB

    info = plsc.get_sparse_core_info()
    NC, NS = info.num_cores, info.num_subcores
    NW = NC * NS
    per_w = n_tasks // NW
    CH = 128
    NBUF = 4
    A = 3
    assert per_w % NBUF == 0

    idx = X.T.reshape(n_tasks, CH).astype(jnp.int32)

    BK = 32768
    nblk = (V + BK - 1) // BK

    def tc_widen_body(tt_ref, out_ref):
        x = tt_ref[...]
        out_ref[:, :D] = x.T * SCALE
        out_ref[:, D:] = jnp.zeros((BK, 128 - D), jnp.float32)

    tc_widen = pl.pallas_call(
        tc_widen_body,
        grid=(nblk,),
        in_specs=[pl.BlockSpec((D, BK), lambda i: (0, i))],
        out_specs=pl.BlockSpec((BK, 128), lambda i: (i, 0)),
        out_shape=jax.ShapeDtypeStruct((V, 128), jnp.float32),
    )
    table16 = tc_widen(table.T).reshape(8 * V, 16)

    mesh = plsc.VectorSubcoreMesh(core_axis_name="c", subcore_axis_name="s")

    @functools.partial(
        pl.kernel,
        mesh=mesh,
        out_type=jax.ShapeDtypeStruct((S, D // 8, RB, 8, 128), jnp.float32),
        compiler_params=pltpu.CompilerParams(
            use_tc_tiling_on_sc=False, needs_layout_passes=False
        ),
        scratch_types=[
            pltpu.VMEM((per_w, CH), jnp.int32),
            pltpu.VMEM((NBUF, 4, CH), jnp.int32),
            pltpu.VMEM((NBUF, 4, CH, 16), jnp.float32),
            pltpu.VMEM((NBUF, 8, 8, 128), jnp.float32),
            pltpu.SemaphoreType.DMA((NBUF,)),
            pltpu.SemaphoreType.DMA((NBUF,)),
        ],
    )
    def sc_kernel(idx_hbm, tab_hbm, out_hbm, idx_v, ebuf, dst8, tbuf, gsem, ssem):
        wid = lax.axis_index("s") * NC + lax.axis_index("c")
        t0 = wid * per_w
        pltpu.sync_copy(idx_hbm.at[pl.ds(t0, per_w)], idx_v)

        iota = lax.iota(jnp.int32, 16)
        tvecs = [iota + (16 * k) for k in range(8)]

        def build_and_fire(task, slot):
            sh = [idx_v[task, pl.ds(16 * g, 16)] * 8 for g in range(8)]
            for c in range(4):
                for g in range(8):
                    ebuf[slot, c, pl.ds(16 * g, 16)] = sh[g] + c
            for c in range(4):
                pltpu.async_copy(
                    tab_hbm.at[ebuf.at[slot, c]], dst8.at[slot, c], gsem.at[slot]
                )

        for c in range(A):
            build_and_fire(c, c)

        @pl.loop(0, per_w, step=NBUF)
        def outer(j):
            for b in range(NBUF):
                jj = j + b
                nxt = jj + A
                b2 = (b + A) % NBUF

                @pl.when(nxt < per_w)
                def _fire_gather():
                    build_and_fire(nxt, b2)

                for c in range(4):
                    pltpu.make_async_copy(
                        tab_hbm.at[ebuf.at[0, 0]], dst8.at[b, c], gsem.at[b]
                    ).wait()

                @pl.when(jj >= NBUF)
                def _drain_store():
                    pltpu.make_async_copy(
                        tbuf.at[b],
                        out_hbm.at[0, pl.ds(0, 8), 0],
                        ssem.at[b],
                    ).wait()

                @plsc.parallel_loop(0, D, unroll=4)
                def _tr(j5):
                    c16 = j5 >> 4
                    jl16 = j5 & 15
                    jh = j5 >> 3
                    jl = j5 & 7
                    cv = jnp.full((16,), 0, jnp.int32) + c16
                    jv = jnp.full((16,), 0, jnp.int32) + jl16
                    for k in range(8):
                        v = plsc.load_gather(dst8.at[b], [cv, tvecs[k], jv])
                        tbuf[b, jh, jl, pl.ds(16 * k, 16)] = v

                t = t0 + jj
                s = t // RB
                rb = lax.rem(t, RB)
                pltpu.async_copy(
                    tbuf.at[b], out_hbm.at[s, pl.ds(0, 8), rb], ssem.at[b]
                )

        for b in range(NBUF):
            pltpu.make_async_copy(
                tbuf.at[b], out_hbm.at[0, pl.ds(0, 8), 0], ssem.at[b]
            ).wait()

    out5 = sc_kernel(idx, table16)
    return out5.transpose(2, 4, 0, 1, 3).reshape(R, S, D)
